# baseline (speedup 1.0000x reference)
import functools

import jax
import jax.numpy as jnp
from jax import lax
from jax.experimental import pallas as pl
from jax.experimental.pallas import tpu as pltpu
from jax.experimental.pallas import tpu_sc as plsc

_NC = 2
_NS = 16
_NW = _NC * _NS
_NBUF = 4


def _emb_call(B, S, D, DP, s_per_w, idx, weight):
    mesh = plsc.VectorSubcoreMesh(core_axis_name="c", subcore_axis_name="s")
    n_per_w = s_per_w * S

    @functools.partial(
        pl.kernel,
        out_type=jax.ShapeDtypeStruct((B, S, DP), jnp.float32),
        mesh=mesh,
        scratch_types=[
            pltpu.VMEM((s_per_w, S), jnp.int32),
            [pltpu.VMEM((S, D), jnp.float32) for _ in range(_NBUF)],
            [pltpu.SemaphoreType.DMA for _ in range(_NBUF)],
        ],
        compiler_params=pltpu.CompilerParams(use_tc_tiling_on_sc=False),
    )
    def emb(idx_hbm, table_hbm, out_hbm, idx_v, rows, gsem):
        wid = lax.axis_index("s") * _NC + lax.axis_index("c")
        seq_base = wid * s_per_w

        pltpu.sync_copy(idx_hbm.at[pl.ds(seq_base, s_per_w)], idx_v)
        for b in range(_NBUF):
            pltpu.async_copy(
                table_hbm.at[idx_v.at[b]], rows[b], gsem[b])

        def outer(jo, carry):
            i0 = jo * _NBUF
            for b in range(_NBUF):
                i = i0 + b
                pltpu.make_async_copy(
                    table_hbm.at[pl.ds(0, S)], rows[b], gsem[b]).wait()
                pltpu.sync_copy(
                    rows[b], out_hbm.at[seq_base + i, :, pl.ds(0, D)])

                @pl.when(i + _NBUF < s_per_w)
                def _():
                    pltpu.async_copy(
                        table_hbm.at[idx_v.at[i + _NBUF]], rows[b], gsem[b])
            return carry

        lax.fori_loop(0, s_per_w // _NBUF, outer, 0)

    return emb(idx, weight)


def kernel(input_ids, weight):
    B, S = input_ids.shape
    V, D = weight.shape
    DP = 128
    idx = input_ids.astype(jnp.int32)

    s_per_w = B // _NW

    out = _emb_call(B, S, D, DP, s_per_w, idx, weight)
    return out[:, :, :D]


# speedup vs baseline: 1.0004x; 1.0004x over previous
import functools

import jax
import jax.numpy as jnp
from jax import lax
from jax.experimental import pallas as pl
from jax.experimental.pallas import tpu as pltpu
from jax.experimental.pallas import tpu_sc as plsc

_NC = 2
_NS = 16
_NW = _NC * _NS
_NBUF = 8


def _emb_call(B, S, D, DP, s_per_w, idx, weight):
    mesh = plsc.VectorSubcoreMesh(core_axis_name="c", subcore_axis_name="s")
    n_per_w = s_per_w * S

    @functools.partial(
        pl.kernel,
        out_type=jax.ShapeDtypeStruct((B, S, DP), jnp.float32),
        mesh=mesh,
        scratch_types=[
            pltpu.VMEM((s_per_w, S), jnp.int32),
            [pltpu.VMEM((S, D), jnp.float32) for _ in range(_NBUF)],
            [pltpu.SemaphoreType.DMA for _ in range(_NBUF)],
        ],
        compiler_params=pltpu.CompilerParams(use_tc_tiling_on_sc=False),
    )
    def emb(idx_hbm, table_hbm, out_hbm, idx_v, rows, gsem):
        wid = lax.axis_index("s") * _NC + lax.axis_index("c")
        seq_base = wid * s_per_w

        pltpu.sync_copy(idx_hbm.at[pl.ds(seq_base, s_per_w)], idx_v)
        for b in range(_NBUF):
            pltpu.async_copy(
                table_hbm.at[idx_v.at[b]], rows[b], gsem[b])

        def outer(jo, carry):
            i0 = jo * _NBUF
            for b in range(_NBUF):
                i = i0 + b
                pltpu.make_async_copy(
                    table_hbm.at[pl.ds(0, S)], rows[b], gsem[b]).wait()
                pltpu.sync_copy(
                    rows[b], out_hbm.at[seq_base + i, :, pl.ds(0, D)])

                @pl.when(i + _NBUF < s_per_w)
                def _():
                    pltpu.async_copy(
                        table_hbm.at[idx_v.at[i + _NBUF]], rows[b], gsem[b])
            return carry

        lax.fori_loop(0, s_per_w // _NBUF, outer, 0)

    return emb(idx, weight)


def kernel(input_ids, weight):
    B, S = input_ids.shape
    V, D = weight.shape
    DP = 128
    idx = input_ids.astype(jnp.int32)

    s_per_w = B // _NW

    out = _emb_call(B, S, D, DP, s_per_w, idx, weight)
    return out[:, :, :D]
